# baseline (speedup 1.0000x reference)
import functools

import jax
import jax.numpy as jnp
from jax import lax
from jax.experimental import pallas as pl
from jax.experimental.pallas import tpu as pltpu
from jax.experimental.pallas import tpu_sc as plsc

N = 100000
D = 64
B = 16384
L = 16
NC = 2
NS = 16
NW = NC * NS
BPW = B // NW
CHUNK = 128
NCHUNK = BPW // CHUNK

_mesh = plsc.VectorSubcoreMesh(core_axis_name="c", subcore_axis_name="s")


@functools.partial(
    pl.kernel,
    out_type=jax.ShapeDtypeStruct((B,), jnp.int32),
    mesh=_mesh,
    compiler_params=pltpu.CompilerParams(needs_layout_passes=False),
    scratch_types=[
        pltpu.VMEM((N,), jnp.int32),
        pltpu.VMEM((B,), jnp.int32),
    ],
)
def _last_writer(idx_hbm, zeros_hbm, m_hbm, pos, xm):
    core = lax.axis_index("c")
    sub = lax.axis_index("s")

    @pl.when(jnp.logical_and(core == 0, sub == 0))
    def _():
        pltpu.sync_copy(zeros_hbm, pos)
        pltpu.sync_copy(idx_hbm, xm)
        lane = lax.iota(jnp.int32, L)

        def scatter_body(c, carry):
            x = xm[pl.ds(c * L, L)]
            j = lane + c * L
            plsc.store_scatter(pos, [x], j)
            r = plsc.load_gather(pos, [x])
            n0 = plsc.all_reduce_population_count(j > r)[0]

            def retry_cond(st):
                return st[0] > 0

            def retry_body(st):
                plsc.store_scatter(pos, [x], j, mask=j > st[1])
                r2 = plsc.load_gather(pos, [x])
                return (plsc.all_reduce_population_count(j > r2)[0], r2)

            lax.while_loop(retry_cond, retry_body, (n0, r))
            return carry

        lax.fori_loop(0, B // L, scatter_body, 0)

        def gather_body(c, carry):
            x = xm[pl.ds(c * L, L)]
            xm[pl.ds(c * L, L)] = plsc.load_gather(pos, [x])
            return carry

        lax.fori_loop(0, B // L, gather_body, 0)
        pltpu.sync_copy(xm, m_hbm)


def _make_gather(nt):
    @functools.partial(
        pl.kernel,
        out_type=tuple(jax.ShapeDtypeStruct((B, D), jnp.float32)
                       for _ in range(nt)),
        mesh=_mesh,
        compiler_params=pltpu.CompilerParams(
            needs_layout_passes=False, use_tc_tiling_on_sc=False),
        scratch_types=[
            pltpu.VMEM((NCHUNK, CHUNK), jnp.int32),
            pltpu.VMEM((CHUNK, D), jnp.float32),
            pltpu.VMEM((CHUNK, D), jnp.float32),
            pltpu.SemaphoreType.DMA,
            pltpu.SemaphoreType.DMA,
            pltpu.SemaphoreType.DMA,
            pltpu.SemaphoreType.DMA,
        ],
    )
    def _gather(m_hbm, *args):
        vs = args[:nt]
        os_ = args[nt:2 * nt]
        m_v, buf_a, buf_b, gsem_a, gsem_b, wsem_a, wsem_b = args[2 * nt:]
        core = lax.axis_index("c")
        sub = lax.axis_index("s")
        wid = sub * NC + core
        base = wid * BPW

        for j in range(NCHUNK):
            pltpu.sync_copy(m_hbm.at[pl.ds(base + j * CHUNK, CHUNK)],
                            m_v.at[j])

        bufs = (buf_a, buf_b)
        gsems = (gsem_a, gsem_b)
        wsems = (wsem_a, wsem_b)
        steps = [(k, j) for k in range(nt) for j in range(NCHUNK)]

        def fire(t):
            k, j = steps[t]
            return pltpu.async_copy(vs[k].at[m_v.at[j]], bufs[t % 2],
                                    gsems[t % 2])

        wcp = [None, None]
        cp = fire(0)
        for t in range(len(steps)):
            if t + 1 < len(steps):
                if wcp[(t + 1) % 2] is not None:
                    wcp[(t + 1) % 2].wait()
                nxt = fire(t + 1)
            else:
                nxt = None
            cp.wait()
            k, j = steps[t]
            wcp[t % 2] = pltpu.async_copy(
                bufs[t % 2], os_[k].at[pl.ds(base + j * CHUNK, CHUNK)],
                wsems[t % 2])
            cp = nxt
        for w in wcp:
            if w is not None:
                w.wait()

    return _gather


_gather2 = _make_gather(2)
_gather3 = _make_gather(3)


def kernel(nodes_memory, crowds_memory, interests_memory, categories_memory,
           brands_memory, values1, values2, values3, values4, values5,
           users_idxs):
    zeros = jnp.zeros((N,), jnp.int32)
    m = _last_writer(users_idxs, zeros)
    o1, o2 = _gather2(m, values1, values2)
    o3, o4, o5 = _gather3(m, values3, values4, values5)
    return (o1, o2, o3, o4, o5)


# speedup vs baseline: 1.0159x; 1.0159x over previous
import functools

import jax
import jax.numpy as jnp
from jax import lax
from jax.experimental import pallas as pl
from jax.experimental.pallas import tpu as pltpu
from jax.experimental.pallas import tpu_sc as plsc

N = 100000
D = 64
B = 16384
L = 16
NC = 2
NS = 16
NW = NC * NS
BPW = B // NW
CHUNK = 128
NCHUNK = BPW // CHUNK

_mesh = plsc.VectorSubcoreMesh(core_axis_name="c", subcore_axis_name="s")


@functools.partial(
    pl.kernel,
    out_type=jax.ShapeDtypeStruct((B,), jnp.int32),
    mesh=_mesh,
    compiler_params=pltpu.CompilerParams(needs_layout_passes=False),
    scratch_types=[
        pltpu.VMEM((N,), jnp.int32),
        pltpu.VMEM((B,), jnp.int32),
    ],
)
def _last_writer(idx_hbm, zeros_hbm, m_hbm, pos, xm):
    core = lax.axis_index("c")
    sub = lax.axis_index("s")

    @pl.when(jnp.logical_and(core == 0, sub == 0))
    def _():
        pltpu.sync_copy(zeros_hbm, pos)
        pltpu.sync_copy(idx_hbm, xm)
        lane = lax.iota(jnp.int32, L)

        def scatter_body(c, carry):
            x = xm[pl.ds(c * L, L)]
            j = lane + c * L
            plsc.store_scatter(pos, [x], j)
            r = plsc.load_gather(pos, [x])
            n0 = plsc.all_reduce_population_count(j > r)[0]

            def retry_cond(st):
                return st[0] > 0

            def retry_body(st):
                plsc.store_scatter(pos, [x], j, mask=j > st[1])
                r2 = plsc.load_gather(pos, [x])
                return (plsc.all_reduce_population_count(j > r2)[0], r2)

            lax.while_loop(retry_cond, retry_body, (n0, r))
            return carry

        lax.fori_loop(0, B // L, scatter_body, 0)

        def gather_body(c, carry):
            x = xm[pl.ds(c * L, L)]
            xm[pl.ds(c * L, L)] = plsc.load_gather(pos, [x])
            return carry

        lax.fori_loop(0, B // L, gather_body, 0)
        pltpu.sync_copy(xm, m_hbm)


@functools.partial(
    pl.kernel,
    out_type=tuple(jax.ShapeDtypeStruct((B, D), jnp.float32) for _ in range(5)),
    mesh=_mesh,
    compiler_params=pltpu.CompilerParams(
        needs_layout_passes=False, use_tc_tiling_on_sc=False),
    scratch_types=[
        pltpu.VMEM((NCHUNK, CHUNK), jnp.int32),
        pltpu.VMEM((CHUNK, D), jnp.float32),
        pltpu.VMEM((CHUNK, D), jnp.float32),
        pltpu.SemaphoreType.DMA,
        pltpu.SemaphoreType.DMA,
        pltpu.SemaphoreType.DMA,
        pltpu.SemaphoreType.DMA,
    ],
)
def _gather5(m_hbm, v1, v2, v3, v4, v5, o1, o2, o3, o4, o5,
             m_v, buf_a, buf_b, gsem_a, gsem_b, wsem_a, wsem_b):
    core = lax.axis_index("c")
    sub = lax.axis_index("s")
    wid = sub * NC + core
    base = wid * BPW

    for j in range(NCHUNK):
        pltpu.sync_copy(m_hbm.at[pl.ds(base + j * CHUNK, CHUNK)], m_v.at[j])

    vs = (v1, v2, v3, v4, v5)
    os_ = (o1, o2, o3, o4, o5)
    bufs = (buf_a, buf_b)
    gsems = (gsem_a, gsem_b)
    wsems = (wsem_a, wsem_b)
    steps = [(k, j) for k in range(5) for j in range(NCHUNK)]

    def fire(t):
        k, j = steps[t]
        return pltpu.async_copy(vs[k].at[m_v.at[j]], bufs[t % 2],
                                gsems[t % 2])

    wcp = [None, None]
    cp = fire(0)
    for t in range(len(steps)):
        if t + 1 < len(steps):
            if wcp[(t + 1) % 2] is not None:
                wcp[(t + 1) % 2].wait()
            nxt = fire(t + 1)
        else:
            nxt = None
        cp.wait()
        k, j = steps[t]
        wcp[t % 2] = pltpu.async_copy(
            bufs[t % 2], os_[k].at[pl.ds(base + j * CHUNK, CHUNK)],
            wsems[t % 2])
        cp = nxt
    for w in wcp:
        if w is not None:
            w.wait()


def kernel(nodes_memory, crowds_memory, interests_memory, categories_memory,
           brands_memory, values1, values2, values3, values4, values5,
           users_idxs):
    zeros = jnp.zeros((N,), jnp.int32)
    m = _last_writer(users_idxs, zeros)
    return _gather5(m, values1, values2, values3, values4, values5)


# speedup vs baseline: 1.0450x; 1.0286x over previous
import functools

import jax
import jax.numpy as jnp
from jax import lax
from jax.experimental import pallas as pl
from jax.experimental.pallas import tpu as pltpu
from jax.experimental.pallas import tpu_sc as plsc

N = 100000
D = 64
B = 16384
L = 16
NC = 2
NS = 16
NW = NC * NS
BPW = B // NW
CHUNK = 128
NCHUNK = BPW // CHUNK

_mesh = plsc.VectorSubcoreMesh(core_axis_name="c", subcore_axis_name="s")


@functools.partial(
    pl.kernel,
    out_type=jax.ShapeDtypeStruct((B,), jnp.int32),
    mesh=_mesh,
    compiler_params=pltpu.CompilerParams(needs_layout_passes=False),
    scratch_types=[
        pltpu.VMEM((N,), jnp.int32),
        pltpu.VMEM((B,), jnp.int32),
    ],
)
def _last_writer(idx_hbm, zeros_hbm, m_hbm, pos, xm):
    core = lax.axis_index("c")
    sub = lax.axis_index("s")

    @pl.when(jnp.logical_and(core == 0, sub == 0))
    def _():
        pltpu.sync_copy(zeros_hbm, pos)
        pltpu.sync_copy(idx_hbm, xm)
        lane = lax.iota(jnp.int32, L)

        def scatter_body(c, carry):
            x = xm[pl.ds(c * L, L)]
            j = lane + c * L
            plsc.store_scatter(pos, [x], j)
            r = plsc.load_gather(pos, [x])
            n0 = plsc.all_reduce_population_count(j > r)[0]

            def retry_cond(st):
                return st[0] > 0

            def retry_body(st):
                plsc.store_scatter(pos, [x], j, mask=j > st[1])
                r2 = plsc.load_gather(pos, [x])
                return (plsc.all_reduce_population_count(j > r2)[0], r2)

            lax.while_loop(retry_cond, retry_body, (n0, r))
            return carry

        lax.fori_loop(0, B // L, scatter_body, 0)

        def gather_body(c, carry):
            x = xm[pl.ds(c * L, L)]
            xm[pl.ds(c * L, L)] = plsc.load_gather(pos, [x])
            return carry

        lax.fori_loop(0, B // L, gather_body, 0)
        pltpu.sync_copy(xm, m_hbm)


@functools.partial(
    pl.kernel,
    out_type=tuple(jax.ShapeDtypeStruct((B, D), jnp.float32) for _ in range(5)),
    mesh=_mesh,
    compiler_params=pltpu.CompilerParams(
        needs_layout_passes=False, use_tc_tiling_on_sc=False),
    scratch_types=[
        pltpu.VMEM((NCHUNK, CHUNK), jnp.int32),
        pltpu.VMEM((CHUNK, D), jnp.float32),
        pltpu.VMEM((CHUNK, D), jnp.float32),
        pltpu.VMEM((CHUNK, D), jnp.float32),
        pltpu.SemaphoreType.DMA,
        pltpu.SemaphoreType.DMA,
        pltpu.SemaphoreType.DMA,
        pltpu.SemaphoreType.DMA,
        pltpu.SemaphoreType.DMA,
        pltpu.SemaphoreType.DMA,
    ],
)
def _gather5(m_hbm, v1, v2, v3, v4, v5, o1, o2, o3, o4, o5,
             m_v, buf_a, buf_b, buf_c,
             gsem_a, gsem_b, gsem_c, wsem_a, wsem_b, wsem_c):
    core = lax.axis_index("c")
    sub = lax.axis_index("s")
    wid = sub * NC + core
    base = wid * BPW

    mcp = [pltpu.async_copy(m_hbm.at[pl.ds(base + j * CHUNK, CHUNK)],
                            m_v.at[j], gsem_a) for j in range(NCHUNK)]
    for cp in mcp:
        cp.wait()

    vs = (v1, v2, v3, v4, v5)
    os_ = (o1, o2, o3, o4, o5)
    bufs = (buf_a, buf_b, buf_c)
    gsems = (gsem_a, gsem_b, gsem_c)
    wsems = (wsem_a, wsem_b, wsem_c)
    steps = [(k, j) for k in range(5) for j in range(NCHUNK)]
    NB = 3

    def fire(t):
        k, j = steps[t]
        return pltpu.async_copy(vs[k].at[m_v.at[j]], bufs[t % NB],
                                gsems[t % NB])

    wcp = [None] * NB
    inflight = [fire(0), fire(1)]
    for t in range(len(steps)):
        if t + 2 < len(steps):
            if wcp[(t + 2) % NB] is not None:
                wcp[(t + 2) % NB].wait()
            inflight.append(fire(t + 2))
        inflight[0].wait()
        inflight.pop(0)
        k, j = steps[t]
        wcp[t % NB] = pltpu.async_copy(
            bufs[t % NB], os_[k].at[pl.ds(base + j * CHUNK, CHUNK)],
            wsems[t % NB])
    for w in wcp:
        if w is not None:
            w.wait()


def kernel(nodes_memory, crowds_memory, interests_memory, categories_memory,
           brands_memory, values1, values2, values3, values4, values5,
           users_idxs):
    zeros = jnp.zeros((N,), jnp.int32)
    m = _last_writer(users_idxs, zeros)
    return _gather5(m, values1, values2, values3, values4, values5)


# speedup vs baseline: 1.0493x; 1.0041x over previous
import functools

import jax
import jax.numpy as jnp
from jax import lax
from jax.experimental import pallas as pl
from jax.experimental.pallas import tpu as pltpu
from jax.experimental.pallas import tpu_sc as plsc

N = 100000
D = 64
B = 16384
L = 16
NC = 2
NS = 16
NW = NC * NS
BPW = B // NW
CHUNK = 128
NCHUNK = BPW // CHUNK

_mesh = plsc.VectorSubcoreMesh(core_axis_name="c", subcore_axis_name="s")


@functools.partial(
    pl.kernel,
    out_type=jax.ShapeDtypeStruct((B,), jnp.int32),
    mesh=_mesh,
    compiler_params=pltpu.CompilerParams(needs_layout_passes=False),
    scratch_types=[
        pltpu.VMEM((N,), jnp.int32),
        pltpu.VMEM((B,), jnp.int32),
    ],
)
def _last_writer(idx_hbm, zeros_hbm, m_hbm, pos, xm):
    core = lax.axis_index("c")
    sub = lax.axis_index("s")

    @pl.when(jnp.logical_and(core == 0, sub == 0))
    def _():
        pltpu.sync_copy(zeros_hbm, pos)
        pltpu.sync_copy(idx_hbm, xm)
        lane = lax.iota(jnp.int32, L)

        def scatter_body(c, carry):
            x = xm[pl.ds(c * L, L)]
            j = lane + c * L
            plsc.store_scatter(pos, [x], j)
            r = plsc.load_gather(pos, [x])
            n0 = plsc.all_reduce_population_count(j > r)[0]

            def retry_cond(st):
                return st[0] > 0

            def retry_body(st):
                plsc.store_scatter(pos, [x], j, mask=j > st[1])
                r2 = plsc.load_gather(pos, [x])
                return (plsc.all_reduce_population_count(j > r2)[0], r2)

            lax.while_loop(retry_cond, retry_body, (n0, r))
            return carry

        lax.fori_loop(0, B // L, scatter_body, 0)

        def gather_body(c, carry):
            x = xm[pl.ds(c * L, L)]
            xm[pl.ds(c * L, L)] = plsc.load_gather(pos, [x])
            return carry

        lax.fori_loop(0, B // L, gather_body, 0)
        pltpu.sync_copy(xm, m_hbm)


@functools.partial(
    pl.kernel,
    out_type=tuple(jax.ShapeDtypeStruct((B, D), jnp.float32) for _ in range(5)),
    mesh=_mesh,
    compiler_params=pltpu.CompilerParams(
        needs_layout_passes=False, use_tc_tiling_on_sc=False),
    scratch_types=[
        pltpu.VMEM((NCHUNK, CHUNK), jnp.int32),
        pltpu.VMEM((CHUNK, D), jnp.float32),
        pltpu.VMEM((CHUNK, D), jnp.float32),
        pltpu.VMEM((CHUNK, D), jnp.float32),
        pltpu.VMEM((CHUNK, D), jnp.float32),
        pltpu.SemaphoreType.DMA,
        pltpu.SemaphoreType.DMA,
        pltpu.SemaphoreType.DMA,
        pltpu.SemaphoreType.DMA,
        pltpu.SemaphoreType.DMA,
        pltpu.SemaphoreType.DMA,
        pltpu.SemaphoreType.DMA,
        pltpu.SemaphoreType.DMA,
    ],
)
def _gather5(m_hbm, v1, v2, v3, v4, v5, o1, o2, o3, o4, o5,
             m_v, buf_a, buf_b, buf_c, buf_d,
             gsem_a, gsem_b, gsem_c, gsem_d,
             wsem_a, wsem_b, wsem_c, wsem_d):
    core = lax.axis_index("c")
    sub = lax.axis_index("s")
    wid = sub * NC + core
    base = wid * BPW

    mcp = [pltpu.async_copy(m_hbm.at[pl.ds(base + j * CHUNK, CHUNK)],
                            m_v.at[j], gsem_a) for j in range(NCHUNK)]
    for cp in mcp:
        cp.wait()

    vs = (v1, v2, v3, v4, v5)
    os_ = (o1, o2, o3, o4, o5)
    bufs = (buf_a, buf_b, buf_c, buf_d)
    gsems = (gsem_a, gsem_b, gsem_c, gsem_d)
    wsems = (wsem_a, wsem_b, wsem_c, wsem_d)
    steps = [(k, j) for k in range(5) for j in range(NCHUNK)]
    NB = 4

    def fire(t):
        k, j = steps[t]
        return pltpu.async_copy(vs[k].at[m_v.at[j]], bufs[t % NB],
                                gsems[t % NB])

    wcp = [None] * NB
    inflight = [fire(0), fire(1), fire(2)]
    for t in range(len(steps)):
        if t + 3 < len(steps):
            if wcp[(t + 3) % NB] is not None:
                wcp[(t + 3) % NB].wait()
            inflight.append(fire(t + 3))
        inflight[0].wait()
        inflight.pop(0)
        k, j = steps[t]
        wcp[t % NB] = pltpu.async_copy(
            bufs[t % NB], os_[k].at[pl.ds(base + j * CHUNK, CHUNK)],
            wsems[t % NB])
    for w in wcp:
        if w is not None:
            w.wait()


def kernel(nodes_memory, crowds_memory, interests_memory, categories_memory,
           brands_memory, values1, values2, values3, values4, values5,
           users_idxs):
    zeros = jnp.zeros((N,), jnp.int32)
    m = _last_writer(users_idxs, zeros)
    return _gather5(m, values1, values2, values3, values4, values5)
